# baseline (device time: 15913 ns/iter reference)
import jax
import jax.numpy as jnp
from jax import lax
from jax.experimental import pallas as pl
from jax.experimental.pallas import tpu as pltpu

N_DEV = 4
B, Sq, Skv, Hq, Dh = 2, 128, 128, 16, 64
H_LOC = Hq // N_DEV
N_CHUNK = 4
ROWS = B * Sq // N_CHUNK


def _body(x_ref, wq_ref, k_ref, v_ref, wo_ref, out_ref,
          send_ref, recv_ref, send_sems, recv_sems):
    my = lax.axis_index("i")
    partner0 = my ^ 1
    partner1 = (N_DEV - 1) - my

    barrier_sem = pltpu.get_barrier_semaphore()
    for nbr in (partner0, partner1):
        pl.semaphore_signal(barrier_sem, inc=1, device_id=(nbr,),
                            device_id_type=pl.DeviceIdType.MESH)
    pl.semaphore_wait(barrier_sem, 2)

    x2 = x_ref[...].reshape(B * Sq, -1).astype(jnp.bfloat16)
    wq = wq_ref[...].astype(jnp.bfloat16)
    q = jnp.dot(x2, wq, preferred_element_type=jnp.float32)

    qb = lax.broadcasted_iota(jnp.int32, (Sq, Skv), 0) // 64
    kb = lax.broadcasted_iota(jnp.int32, (Sq, Skv), 1) // 64
    mask = (qb == kb) | (kb == 0) | ((qb + kb) % 3 == 0)

    wo = wo_ref[...].astype(jnp.bfloat16)

    def exch(p, c, partner):
        return pltpu.make_async_remote_copy(
            src_ref=send_ref.at[p, c],
            dst_ref=recv_ref.at[p, c],
            send_sem=send_sems.at[p, c],
            recv_sem=recv_sems.at[p, c],
            device_id=(partner,),
            device_id_type=pl.DeviceIdType.MESH,
        )

    def out_slot(c):
        b, hf = divmod(c, Sq // ROWS)
        return out_ref.at[b, pl.ds(hf * ROWS, ROWS), :]

    def compute_chunk(c):
        b, hf = divmod(c, Sq // ROWS)
        ctxs = []
        for h in range(H_LOC):
            qch = q[b * Sq + hf * ROWS:b * Sq + (hf + 1) * ROWS,
                    h * Dh:(h + 1) * Dh].astype(jnp.bfloat16)
            kbh = k_ref[b, :, h, :].astype(jnp.bfloat16)
            vbh = v_ref[b, :, h, :].astype(jnp.bfloat16)
            s = jnp.dot(qch, kbh.T, preferred_element_type=jnp.float32) * 0.125
            e = jnp.where(mask[hf * ROWS:(hf + 1) * ROWS, :], jnp.exp(s), 0.0)
            ctx = jnp.dot(e.astype(jnp.bfloat16), vbh,
                          preferred_element_type=jnp.float32)
            ctxs.append(ctx / jnp.sum(e, axis=-1, keepdims=True))
        ctx_c = jnp.concatenate(ctxs, axis=1).astype(jnp.bfloat16)
        return jnp.dot(ctx_c, wo, preferred_element_type=jnp.float32)

    rdmas0, rdmas1 = {}, {}

    def finish_phase0(c):
        rdmas0[c].wait_recv()
        acc = out_slot(c)[...] + recv_ref[0, c, ...].astype(jnp.float32)
        out_slot(c)[...] = acc
        send_ref[1, c, ...] = acc.astype(jnp.bfloat16)
        rdmas1[c] = exch(1, c, partner1)
        rdmas1[c].start()

    for c in range(N_CHUNK):
        acc = compute_chunk(c)
        out_slot(c)[...] = acc
        send_ref[0, c, ...] = acc.astype(jnp.bfloat16)
        rdmas0[c] = exch(0, c, partner0)
        rdmas0[c].start()
        if c >= 2:
            finish_phase0(c - 2)
    finish_phase0(N_CHUNK - 2)
    finish_phase0(N_CHUNK - 1)

    for c in range(N_CHUNK):
        rdmas1[c].wait_recv()
        out_slot(c)[...] += recv_ref[1, c, ...].astype(jnp.float32)

    for c in range(N_CHUNK):
        rdmas0[c].wait_send()
        rdmas1[c].wait_send()


def kernel(x, Wq, K_ext, V_ext, Wo):
    my = lax.axis_index("i")
    k_sh = lax.dynamic_slice_in_dim(K_ext, my * H_LOC, H_LOC, axis=2)
    v_sh = lax.dynamic_slice_in_dim(V_ext, my * H_LOC, H_LOC, axis=2)
    return pl.pallas_call(
        _body,
        out_shape=jax.ShapeDtypeStruct((B, Sq, Wo.shape[1]), jnp.float32),
        in_specs=[pl.BlockSpec(memory_space=pltpu.VMEM)] * 5,
        out_specs=pl.BlockSpec(memory_space=pltpu.VMEM),
        scratch_shapes=[
            pltpu.VMEM((2, N_CHUNK, ROWS, Wo.shape[1]), jnp.bfloat16),
            pltpu.VMEM((2, N_CHUNK, ROWS, Wo.shape[1]), jnp.bfloat16),
            pltpu.SemaphoreType.DMA((2, N_CHUNK)),
            pltpu.SemaphoreType.DMA((2, N_CHUNK)),
        ],
        compiler_params=pltpu.CompilerParams(collective_id=0),
    )(x, Wq, k_sh, v_sh, Wo)


# device time: 14653 ns/iter; 1.0860x vs baseline; 1.0860x over previous
import jax
import jax.numpy as jnp
from jax import lax
from jax.experimental import pallas as pl
from jax.experimental.pallas import tpu as pltpu

N_DEV = 4
B, Sq, Skv, Hq, Dh = 2, 128, 128, 16, 64
H_LOC = Hq // N_DEV
CHUNKS = ((0, 0, 128), (1, 0, 64), (1, 64, 64))
N_CHUNK = len(CHUNKS)
MAX_ROWS = max(c[2] for c in CHUNKS)


def _body(x_ref, wq_ref, k_ref, v_ref, wo_ref, out_ref,
          send_ref, recv_ref, send_sems, recv_sems):
    my = lax.axis_index("i")
    partner0 = my ^ 1
    partner1 = (N_DEV - 1) - my

    barrier_sem = pltpu.get_barrier_semaphore()
    for nbr in (partner0, partner1):
        pl.semaphore_signal(barrier_sem, inc=1, device_id=(nbr,),
                            device_id_type=pl.DeviceIdType.MESH)

    x2 = x_ref[...].reshape(B * Sq, -1).astype(jnp.bfloat16)
    wq = wq_ref[...].astype(jnp.bfloat16)
    q = jnp.dot(x2, wq, preferred_element_type=jnp.float32)

    qb = lax.broadcasted_iota(jnp.int32, (Sq, Skv), 0) // 64
    kb = lax.broadcasted_iota(jnp.int32, (Sq, Skv), 1) // 64
    mask = (qb == kb) | (kb == 0) | ((qb + kb) % 3 == 0)

    wo = wo_ref[...].astype(jnp.bfloat16)

    def exch(p, c, partner):
        rows = CHUNKS[c][2]
        return pltpu.make_async_remote_copy(
            src_ref=send_ref.at[p, c, pl.ds(0, rows)],
            dst_ref=recv_ref.at[p, c, pl.ds(0, rows)],
            send_sem=send_sems.at[p, c],
            recv_sem=recv_sems.at[p, c],
            device_id=(partner,),
            device_id_type=pl.DeviceIdType.MESH,
        )

    def out_slot(c):
        b, r0, rows = CHUNKS[c]
        return out_ref.at[b, pl.ds(r0, rows), :]

    def compute_chunk(c):
        b, r0, rows = CHUNKS[c]
        ctxs = []
        for h in range(H_LOC):
            qch = q[b * Sq + r0:b * Sq + r0 + rows,
                    h * Dh:(h + 1) * Dh].astype(jnp.bfloat16)
            kbh = k_ref[b, :, h, :].astype(jnp.bfloat16)
            vbh = v_ref[b, :, h, :].astype(jnp.bfloat16)
            s = jnp.dot(qch, kbh.T, preferred_element_type=jnp.float32) * 0.125
            e = jnp.where(mask[r0:r0 + rows, :], jnp.exp(s), 0.0)
            ctx = jnp.dot(e.astype(jnp.bfloat16), vbh,
                          preferred_element_type=jnp.float32)
            ctxs.append(ctx / jnp.sum(e, axis=-1, keepdims=True))
        ctx_c = jnp.concatenate(ctxs, axis=1).astype(jnp.bfloat16)
        return jnp.dot(ctx_c, wo, preferred_element_type=jnp.float32)

    rdmas0, rdmas1 = {}, {}

    def finish_phase0(c):
        rows = CHUNKS[c][2]
        rdmas0[c].wait_recv()
        acc = out_slot(c)[...] + recv_ref[0, c, pl.ds(0, rows)].astype(jnp.float32)
        out_slot(c)[...] = acc
        send_ref[1, c, pl.ds(0, rows)] = acc.astype(jnp.bfloat16)
        rdmas1[c] = exch(1, c, partner1)
        rdmas1[c].start()

    for c in range(N_CHUNK):
        acc = compute_chunk(c)
        out_slot(c)[...] = acc
        send_ref[0, c, pl.ds(0, CHUNKS[c][2])] = acc.astype(jnp.bfloat16)
        if c == 0:
            pl.semaphore_wait(barrier_sem, 2)
        rdmas0[c] = exch(0, c, partner0)
        rdmas0[c].start()
    for c in range(N_CHUNK):
        finish_phase0(c)

    for c in range(N_CHUNK):
        rows = CHUNKS[c][2]
        rdmas1[c].wait_recv()
        out_slot(c)[...] += recv_ref[1, c, pl.ds(0, rows)].astype(jnp.float32)

    for c in range(N_CHUNK):
        rdmas0[c].wait_send()
        rdmas1[c].wait_send()


def kernel(x, Wq, K_ext, V_ext, Wo):
    my = lax.axis_index("i")
    k_sh = lax.dynamic_slice_in_dim(K_ext, my * H_LOC, H_LOC, axis=2)
    v_sh = lax.dynamic_slice_in_dim(V_ext, my * H_LOC, H_LOC, axis=2)
    return pl.pallas_call(
        _body,
        out_shape=jax.ShapeDtypeStruct((B, Sq, Wo.shape[1]), jnp.float32),
        in_specs=[pl.BlockSpec(memory_space=pltpu.VMEM)] * 5,
        out_specs=pl.BlockSpec(memory_space=pltpu.VMEM),
        scratch_shapes=[
            pltpu.VMEM((2, N_CHUNK, MAX_ROWS, Wo.shape[1]), jnp.bfloat16),
            pltpu.VMEM((2, N_CHUNK, MAX_ROWS, Wo.shape[1]), jnp.bfloat16),
            pltpu.SemaphoreType.DMA((2, N_CHUNK)),
            pltpu.SemaphoreType.DMA((2, N_CHUNK)),
        ],
        compiler_params=pltpu.CompilerParams(collective_id=0),
    )(x, Wq, k_sh, v_sh, Wo)
